# baseline (speedup 1.0000x reference)
import functools

import jax
import jax.numpy as jnp
from jax import lax
from jax.experimental import pallas as pl
from jax.experimental.pallas import tpu as pltpu
from jax.experimental.pallas import tpu_sc as plsc

_R = 200


def _hingcn_body(adj0, adj1, feats, wp0, wp1, wa0, wa1, ws0, ws1, av,
                 beta_out, agg_out, of):
    i = pl.program_id(0)

    @pl.when(i == 0)
    def _prep():
        of[...] = jnp.dot(feats[...], wp1[...],
                          preferred_element_type=jnp.float32)

    fblk = feats[pl.ds(i * _R, _R), :]
    ff = jnp.dot(fblk, wp0[...], preferred_element_type=jnp.float32)

    hs = []
    for adj, wa, ws in ((adj0, wa0, ws0), (adj1, wa1, ws1)):
        neigh = jnp.dot(adj[...], of[...],
                        preferred_element_type=jnp.float32)
        h = jnp.dot(neigh, wa[...], preferred_element_type=jnp.float32)
        h = h + jnp.dot(ff, ws[...], preferred_element_type=jnp.float32)
        hs.append(jnp.maximum(h, 0.0))
    h0, h1 = hs

    s0 = jnp.sum(jnp.tanh(h0) * av[...], axis=1, keepdims=True)
    s1 = jnp.sum(jnp.tanh(h1) * av[...], axis=1, keepdims=True)
    m = jnp.maximum(s0, s1)
    e0 = jnp.exp(s0 - m)
    e1 = jnp.exp(s1 - m)
    inv = 1.0 / (e0 + e1)
    b0 = e0 * inv
    b1 = e1 * inv

    beta_out[...] = jnp.concatenate([b0, b1], axis=1)
    agg_out[...] = b0 * h0 + b1 * h1


def _fc_body(g, wfc, bfc, out):
    out[...] = jnp.dot(g[...], wfc[...],
                       preferred_element_type=jnp.float32) + bfc[...]


def _sc_gather(table, idx):
    n, d = table.shape
    b = idx.shape[0]
    info = plsc.get_sparse_core_info()
    nw = info.num_cores * info.num_subcores
    bpw = b // nw
    mesh = plsc.VectorSubcoreMesh(core_axis_name="c", subcore_axis_name="s")

    @functools.partial(
        pl.kernel, mesh=mesh,
        out_type=jax.ShapeDtypeStruct((b, d), jnp.float32),
        scratch_types=[
            pltpu.VMEM((bpw,), jnp.int32),
            pltpu.VMEM((bpw, d), jnp.float32),
            pltpu.SemaphoreType.DMA,
        ],
    )
    def gk(table_hbm, idx_hbm, out_hbm, idx_v, rows_v, sem):
        wid = lax.axis_index("s") * info.num_cores + lax.axis_index("c")
        base = wid * bpw
        pltpu.sync_copy(idx_hbm.at[pl.ds(base, bpw)], idx_v)
        pltpu.async_copy(table_hbm.at[idx_v], rows_v, sem).wait()
        pltpu.sync_copy(rows_v, out_hbm.at[pl.ds(base, bpw)])

    return gk(table, idx)


def kernel(ids, feats, adjs_0, adjs_1, W_prep0, W_prep1, W_agg_0, W_agg_1,
           W_self_0, W_self_1, att_vec, W_fc, b_fc):
    n, d_feat = feats.shape
    prep = W_prep0.shape[1]
    hid = W_agg_0.shape[1]
    ncls = W_fc.shape[1]
    g = n // _R

    av2 = att_vec.reshape(1, hid)
    bfc2 = b_fc.reshape(1, ncls)

    full = lambda shape: pl.BlockSpec(shape, lambda i: (0, 0))
    beta_t, agg = pl.pallas_call(
        _hingcn_body,
        grid=(g,),
        in_specs=[
            pl.BlockSpec((_R, n), lambda i: (i, 0)),
            pl.BlockSpec((_R, n), lambda i: (i, 0)),
            full((n, d_feat)),
            full((d_feat, prep)), full((d_feat, prep)),
            full((prep, hid)), full((prep, hid)),
            full((prep, hid)), full((prep, hid)),
            full((1, hid)),
        ],
        out_specs=[
            pl.BlockSpec((_R, 2), lambda i: (i, 0)),
            pl.BlockSpec((_R, hid), lambda i: (i, 0)),
        ],
        out_shape=[
            jax.ShapeDtypeStruct((n, 2), jnp.float32),
            jax.ShapeDtypeStruct((n, hid), jnp.float32),
        ],
        scratch_shapes=[pltpu.VMEM((n, prep), jnp.float32)],
        compiler_params=pltpu.CompilerParams(
            dimension_semantics=("arbitrary",)),
    )(adjs_0, adjs_1, feats, W_prep0, W_prep1, W_agg_0, W_agg_1,
      W_self_0, W_self_1, av2)

    gathered = _sc_gather(agg, ids.astype(jnp.int32))

    logits = pl.pallas_call(
        _fc_body,
        out_shape=jax.ShapeDtypeStruct((ids.shape[0], ncls), jnp.float32),
    )(gathered, W_fc, bfc2)

    return (logits, beta_t.T)


# speedup vs baseline: 1.1040x; 1.1040x over previous
import functools

import jax
import jax.numpy as jnp
from jax import lax
from jax.experimental import pallas as pl
from jax.experimental.pallas import tpu as pltpu
from jax.experimental.pallas import tpu_sc as plsc

_R = 200


def _hingcn_body(adj0, adj1, feats, wp0, wp1, wa0, wa1, ws0, ws1, av,
                 beta_out, agg_out, of):
    i = pl.program_id(0)

    @pl.when(i == 0)
    def _prep():
        of[...] = jnp.dot(feats[...], wp1[...],
                          preferred_element_type=jnp.float32)

    fblk = feats[pl.ds(i * _R, _R), :]
    ff = jnp.dot(fblk, wp0[...], preferred_element_type=jnp.float32)

    of_b = of[...].astype(jnp.bfloat16)
    hs = []
    for adj, wa, ws in ((adj0, wa0, ws0), (adj1, wa1, ws1)):
        neigh = lax.dot_general(adj[...].astype(jnp.bfloat16), of_b,
                                (((1,), (0,)), ((), ())),
                                preferred_element_type=jnp.float32)
        h = jnp.dot(neigh, wa[...], preferred_element_type=jnp.float32)
        h = h + jnp.dot(ff, ws[...], preferred_element_type=jnp.float32)
        hs.append(jnp.maximum(h, 0.0))
    h0, h1 = hs

    s0 = jnp.sum(jnp.tanh(h0) * av[...], axis=1, keepdims=True)
    s1 = jnp.sum(jnp.tanh(h1) * av[...], axis=1, keepdims=True)
    m = jnp.maximum(s0, s1)
    e0 = jnp.exp(s0 - m)
    e1 = jnp.exp(s1 - m)
    inv = 1.0 / (e0 + e1)
    b0 = e0 * inv
    b1 = e1 * inv

    beta_out[...] = jnp.concatenate([b0, b1], axis=1)
    agg_out[...] = b0 * h0 + b1 * h1


def _fc_body(g, wfc, bfc, out):
    out[...] = jnp.dot(g[...], wfc[...],
                       preferred_element_type=jnp.float32) + bfc[...]


def _sc_gather(table, idx):
    n, d = table.shape
    b = idx.shape[0]
    info = plsc.get_sparse_core_info()
    nw = info.num_cores * info.num_subcores
    bpw = b // nw
    mesh = plsc.VectorSubcoreMesh(core_axis_name="c", subcore_axis_name="s")

    @functools.partial(
        pl.kernel, mesh=mesh,
        out_type=jax.ShapeDtypeStruct((b, d), jnp.float32),
        scratch_types=[
            pltpu.VMEM((bpw,), jnp.int32),
            pltpu.VMEM((bpw, d), jnp.float32),
            pltpu.SemaphoreType.DMA,
        ],
    )
    def gk(table_hbm, idx_hbm, out_hbm, idx_v, rows_v, sem):
        wid = lax.axis_index("s") * info.num_cores + lax.axis_index("c")
        base = wid * bpw
        pltpu.sync_copy(idx_hbm.at[pl.ds(base, bpw)], idx_v)
        pltpu.async_copy(table_hbm.at[idx_v], rows_v, sem).wait()
        pltpu.sync_copy(rows_v, out_hbm.at[pl.ds(base, bpw)])

    return gk(table, idx)


def kernel(ids, feats, adjs_0, adjs_1, W_prep0, W_prep1, W_agg_0, W_agg_1,
           W_self_0, W_self_1, att_vec, W_fc, b_fc):
    n, d_feat = feats.shape
    prep = W_prep0.shape[1]
    hid = W_agg_0.shape[1]
    ncls = W_fc.shape[1]
    g = n // _R

    av2 = att_vec.reshape(1, hid)
    bfc2 = b_fc.reshape(1, ncls)

    full = lambda shape: pl.BlockSpec(shape, lambda i: (0, 0))
    beta_t, agg = pl.pallas_call(
        _hingcn_body,
        grid=(g,),
        in_specs=[
            pl.BlockSpec((_R, n), lambda i: (i, 0)),
            pl.BlockSpec((_R, n), lambda i: (i, 0)),
            full((n, d_feat)),
            full((d_feat, prep)), full((d_feat, prep)),
            full((prep, hid)), full((prep, hid)),
            full((prep, hid)), full((prep, hid)),
            full((1, hid)),
        ],
        out_specs=[
            pl.BlockSpec((_R, 2), lambda i: (i, 0)),
            pl.BlockSpec((_R, hid), lambda i: (i, 0)),
        ],
        out_shape=[
            jax.ShapeDtypeStruct((n, 2), jnp.float32),
            jax.ShapeDtypeStruct((n, hid), jnp.float32),
        ],
        scratch_shapes=[pltpu.VMEM((n, prep), jnp.float32)],
        compiler_params=pltpu.CompilerParams(
            dimension_semantics=("arbitrary",)),
    )(adjs_0, adjs_1, feats, W_prep0, W_prep1, W_agg_0, W_agg_1,
      W_self_0, W_self_1, av2)

    gathered = _sc_gather(agg, ids.astype(jnp.int32))

    logits = pl.pallas_call(
        _fc_body,
        out_shape=jax.ShapeDtypeStruct((ids.shape[0], ncls), jnp.float32),
    )(gathered, W_fc, bfc2)

    return (logits, beta_t.T)
